# Initial kernel scaffold; baseline (speedup 1.0000x reference)
import functools
import jax
import jax.numpy as jnp
from jax import lax
from jax.experimental import pallas as pl
from jax.experimental.pallas import tpu as pltpu
from jax.experimental.pallas import tpu_sc as plsc

_VOCAB = 1000000
_EMBED = 64
_MAXLEN = 200
_B = 4096
_L = 200
_N = _B * _L

_NC = 2
_NS = 16
_NW = _NC * _NS
_PER_W = _N // _NW
_CHUNK = 800
_NCHUNK = _PER_W // _CHUNK
_EPS = 1e-5


def _rsqrt_newton(v):
    bits = lax.bitcast_convert_type(v, jnp.int32)
    y = lax.bitcast_convert_type(
        jnp.int32(0x5F3759DF) - lax.shift_right_logical(bits, 1), jnp.float32)
    for _ in range(3):
        y = y * (1.5 - 0.5 * v * y * y)
    return y


def _emb_body(ids_hbm, tok_hbm, pos_hbm, gam_hbm, bet_hbm, out_hbm,
              idx_v, rows_v, pos_v, gb_v, sem):
    wid = lax.axis_index("s") * _NC + lax.axis_index("c")
    base = wid * _PER_W

    pltpu.sync_copy(pos_hbm, pos_v)
    pltpu.sync_copy(gam_hbm, gb_v.at[0])
    pltpu.sync_copy(bet_hbm, gb_v.at[1])

    gs = [gb_v[0, pl.ds(16 * j, 16)] for j in range(4)]
    bs = [gb_v[1, pl.ds(16 * j, 16)] for j in range(4)]

    def chunk_body(ci, carry):
        cb = base + ci * _CHUNK
        pltpu.sync_copy(ids_hbm.at[pl.ds(cb, _CHUNK)], idx_v)
        pltpu.async_copy(tok_hbm.at[idx_v], rows_v, sem).wait()

        def row_body(r, rcarry):
            p = lax.rem(r, _MAXLEN)
            xs = []
            for j in range(4):
                x = rows_v[r, pl.ds(16 * j, 16)] + pos_v[p, pl.ds(16 * j, 16)]
                xs.append(x)
            s = (xs[0] + xs[1]) + (xs[2] + xs[3])
            q = ((xs[0] * xs[0] + xs[1] * xs[1])
                 + (xs[2] * xs[2] + xs[3] * xs[3]))
            tot = jnp.sum(s)
            totq = jnp.sum(q)
            mean = tot * (1.0 / _EMBED)
            var = totq * (1.0 / _EMBED) - mean * mean
            vv = jnp.broadcast_to(var + _EPS, (16,))
            rstd = _rsqrt_newton(vv)
            meanv = jnp.broadcast_to(mean, (16,))
            for j in range(4):
                rows_v[r, pl.ds(16 * j, 16)] = (
                    (xs[j] - meanv) * rstd * gs[j] + bs[j])
            return rcarry

        lax.fori_loop(0, _CHUNK, row_body, 0)
        pltpu.sync_copy(rows_v, out_hbm.at[pl.ds(cb, _CHUNK)])
        return carry

    lax.fori_loop(0, _NCHUNK, chunk_body, 0)


_emb_kernel = functools.partial(
    pl.kernel,
    mesh=plsc.VectorSubcoreMesh(core_axis_name="c", subcore_axis_name="s"),
    out_type=jax.ShapeDtypeStruct((_N, _EMBED), jnp.float32),
    scratch_types=[
        pltpu.VMEM((_CHUNK,), jnp.int32),
        pltpu.VMEM((_CHUNK, _EMBED), jnp.float32),
        pltpu.VMEM((_MAXLEN, _EMBED), jnp.float32),
        pltpu.VMEM((2, _EMBED), jnp.float32),
        pltpu.SemaphoreType.DMA,
    ],
)(_emb_body)


@jax.jit
def kernel(input_ids, token_table, pos_table, gamma, beta):
    ids = input_ids.reshape(-1).astype(jnp.int32)
    out = _emb_kernel(ids, token_table, pos_table, gamma, beta)
    return out.reshape(_B, _L, _EMBED)


# speedup vs baseline: 2.4928x; 2.4928x over previous
import functools
import jax
import jax.numpy as jnp
from jax import lax
from jax.experimental import pallas as pl
from jax.experimental.pallas import tpu as pltpu
from jax.experimental.pallas import tpu_sc as plsc

_VOCAB = 1000000
_EMBED = 64
_MAXLEN = 200
_B = 4096
_L = 200
_N = _B * _L

_NC = 2
_NS = 16
_NW = _NC * _NS
_PER_W = _N // _NW
_CHUNK = 800
_NCHUNK = _PER_W // _CHUNK
_EPS = 1e-5

_GATHER_DNUMS = lax.GatherDimensionNumbers(
    offset_dims=(), collapsed_slice_dims=(0,), start_index_map=(0,))


def _lane_gather(v, perm):
    return lax.gather(v, perm[:, None], _GATHER_DNUMS, slice_sizes=(1,),
                      mode=lax.GatherScatterMode.PROMISE_IN_BOUNDS)


def _rsqrt_newton(v):
    bits = lax.bitcast_convert_type(v, jnp.int32)
    y = lax.bitcast_convert_type(
        jnp.int32(0x5F3759DF) - lax.shift_right_logical(bits, 1), jnp.float32)
    for _ in range(3):
        y = y * (1.5 - 0.5 * v * y * y)
    return y


def _emb_body(ids_hbm, tok_hbm, pos_hbm, gam_hbm, bet_hbm, out_hbm,
              idx_a, idx_b, rows_a, rows_b, pos_v, gb_v, gsem_a, gsem_b):
    wid = lax.axis_index("s") * _NC + lax.axis_index("c")
    base = wid * _PER_W

    pltpu.sync_copy(pos_hbm, pos_v)
    pltpu.sync_copy(gam_hbm, gb_v.at[0])
    pltpu.sync_copy(bet_hbm, gb_v.at[1])

    gs = [gb_v[0, pl.ds(16 * j, 16)] for j in range(4)]
    bs = [gb_v[1, pl.ds(16 * j, 16)] for j in range(4)]
    lanes = lax.iota(jnp.int32, 16)

    def start_gather(ci, idx_v, rows_v, sem):
        cb = base + ci * _CHUNK
        pltpu.sync_copy(ids_hbm.at[pl.ds(cb, _CHUNK)], idx_v)
        pltpu.async_copy(tok_hbm.at[idx_v], rows_v, sem)

    def process(ci, idx_v, rows_v, sem):
        pltpu.make_async_copy(tok_hbm.at[idx_v], rows_v, sem).wait()

        @plsc.parallel_loop(0, _CHUNK, step=1, unroll=4)
        def _row(r):
            p = lax.rem(r, _MAXLEN)
            xs = []
            for j in range(4):
                x = rows_v[r, pl.ds(16 * j, 16)] + pos_v[p, pl.ds(16 * j, 16)]
                xs.append(x)
            s = (xs[0] + xs[1]) + (xs[2] + xs[3])
            q = ((xs[0] * xs[0] + xs[1] * xs[1])
                 + (xs[2] * xs[2] + xs[3] * xs[3]))
            for sh in (8, 4, 2, 1):
                perm = lax.bitwise_xor(lanes, jnp.int32(sh))
                s = s + _lane_gather(s, perm)
                q = q + _lane_gather(q, perm)
            mean = s * (1.0 / _EMBED)
            var = q * (1.0 / _EMBED) - mean * mean
            rstd = _rsqrt_newton(var + _EPS)
            for j in range(4):
                rows_v[r, pl.ds(16 * j, 16)] = (
                    (xs[j] - mean) * rstd * gs[j] + bs[j])

        pltpu.sync_copy(rows_v, out_hbm.at[pl.ds(base + ci * _CHUNK, _CHUNK)])

    start_gather(0, idx_a, rows_a, gsem_a)

    def pair_body(jj, carry):
        c0 = 2 * jj
        start_gather(c0 + 1, idx_b, rows_b, gsem_b)
        process(c0, idx_a, rows_a, gsem_a)

        @pl.when(jj < _NCHUNK // 2 - 1)
        def _():
            start_gather(c0 + 2, idx_a, rows_a, gsem_a)

        process(c0 + 1, idx_b, rows_b, gsem_b)
        return carry

    lax.fori_loop(0, _NCHUNK // 2, pair_body, 0)


_emb_kernel = functools.partial(
    pl.kernel,
    mesh=plsc.VectorSubcoreMesh(core_axis_name="c", subcore_axis_name="s"),
    out_type=jax.ShapeDtypeStruct((_N, _EMBED), jnp.float32),
    scratch_types=[
        pltpu.VMEM((_CHUNK,), jnp.int32),
        pltpu.VMEM((_CHUNK,), jnp.int32),
        pltpu.VMEM((_CHUNK, _EMBED), jnp.float32),
        pltpu.VMEM((_CHUNK, _EMBED), jnp.float32),
        pltpu.VMEM((_MAXLEN, _EMBED), jnp.float32),
        pltpu.VMEM((2, _EMBED), jnp.float32),
        pltpu.SemaphoreType.DMA,
        pltpu.SemaphoreType.DMA,
    ],
    compiler_params=pltpu.CompilerParams(use_tc_tiling_on_sc=False),
)(_emb_body)


@jax.jit
def kernel(input_ids, token_table, pos_table, gamma, beta):
    ids = input_ids.reshape(-1).astype(jnp.int32)
    out = _emb_kernel(ids, token_table, pos_table, gamma, beta)
    return out.reshape(_B, _L, _EMBED)
